# Initial kernel scaffold; baseline (speedup 1.0000x reference)
import jax
import jax.numpy as jnp
from jax.experimental import pallas as pl


def kernel(x, bridge, w_up, b_up, w_uc, b_uc, ln_g, ln_b, w1, b1, w2, b2):
    raise NotImplementedError("write your pallas kernel here")


# speedup vs baseline: 1.1020x; 1.1020x over previous
import functools

import jax
import jax.numpy as jnp
from jax.experimental import pallas as pl
from jax.experimental.pallas import tpu as pltpu


def _fused_kernel(x_ref, br_ref, wup_ref, bup_ref, wuc_ref, buc_ref,
                  g_ref, bln_ref, w1_ref, b1_ref, w2_ref, b2_ref,
                  o_ref, up_sc, cat_sc, tpad_sc, *, slope, eps):
    f32 = jnp.float32
    bf16 = jnp.bfloat16
    _, H, W, Cin = x_ref.shape
    _, Ho, Wo, C = o_ref.shape
    Ctot = cat_sc.shape[-1]
    P = Ho * Wo

    cat_sc[...] = jnp.zeros_like(cat_sc)
    tpad_sc[...] = jnp.zeros_like(tpad_sc)

    x2d = x_ref[0].reshape(H * W, Cin).astype(bf16)
    yup = (jnp.dot(x2d, wup_ref[...], preferred_element_type=f32)
           + bup_ref[...])
    for di in range(2):
        up_sc[:, di] = (yup[:, di * 2 * C:(di + 1) * 2 * C]
                        .reshape(H, W, 2 * C).astype(bf16))
    cat_sc[1:Ho + 1, 1:Wo + 1, 0:C] = up_sc[...].reshape(Ho, Wo, C)
    cat_sc[1:Ho + 1, 1:Wo + 1, C:Ctot] = br_ref[0].astype(bf16)

    def conv3x3(src, Cc, w_ref, b_ref):
        acc = jnp.broadcast_to(b_ref[...], (P, C)).astype(f32)
        for ky in range(3):
            for kx in range(3):
                k = ky * 3 + kx
                a = src[ky:ky + Ho, kx:kx + Wo, :].reshape(P, Cc)
                acc = acc + jnp.dot(a, w_ref[k * Cc:(k + 1) * Cc, :],
                                    preferred_element_type=f32)
        return acc

    y = conv3x3(cat_sc, Ctot, wuc_ref, buc_ref)
    o_ref[0] = y.reshape(Ho, Wo, C)

    mu = jnp.mean(y, axis=-1, keepdims=True)
    var = jnp.mean((y - mu) ** 2, axis=-1, keepdims=True)
    t = (y - mu) * jax.lax.rsqrt(var + eps) * g_ref[...] + bln_ref[...]
    tpad_sc[1:Ho + 1, 1:Wo + 1, :] = t.reshape(Ho, Wo, C).astype(bf16)

    h = conv3x3(tpad_sc, C, w1_ref, b1_ref)
    h = jnp.where(h >= 0, h, h * slope)
    tpad_sc[1:Ho + 1, 1:Wo + 1, :] = h.reshape(Ho, Wo, C).astype(bf16)
    h = conv3x3(tpad_sc, C, w2_ref, b2_ref)

    o_ref[0] = (o_ref[0] + h.reshape(Ho, Wo, C)).astype(o_ref.dtype)


def kernel(x, bridge, w_up, b_up, w_uc, b_uc, ln_g, ln_b, w1, b1, w2, b2):
    N, H, W, Cin = x.shape
    C = w_up.shape[-1]
    Cb = bridge.shape[-1]
    Ho, Wo = 2 * H, 2 * W
    Ctot = C + Cb
    bf16 = jnp.bfloat16

    wup_p = jnp.transpose(w_up, (1, 0, 2)).reshape(Cin, 4 * C).astype(bf16)
    bup_p = jnp.tile(b_up, 4).reshape(1, 4 * C)
    wuc_p = w_uc.reshape(9 * Ctot, C).astype(bf16)
    buc_p = b_uc.reshape(1, C)
    g_p = ln_g.reshape(1, C)
    bln_p = ln_b.reshape(1, C)
    w1_p = w1.reshape(9 * C, C).astype(bf16)
    b1_p = b1.reshape(1, C)
    w2_p = w2.reshape(9 * C, C).astype(bf16)
    b2_p = b2.reshape(1, C)

    return pl.pallas_call(
        functools.partial(_fused_kernel, slope=0.2, eps=1e-5),
        out_shape=jax.ShapeDtypeStruct((N, Ho, Wo, C), x.dtype),
        grid=(N,),
        in_specs=[
            pl.BlockSpec((1, H, W, Cin), lambda n: (n, 0, 0, 0)),
            pl.BlockSpec((1, Ho, Wo, Cb), lambda n: (n, 0, 0, 0)),
            pl.BlockSpec((Cin, 4 * C), lambda n: (0, 0)),
            pl.BlockSpec((1, 4 * C), lambda n: (0, 0)),
            pl.BlockSpec((9 * Ctot, C), lambda n: (0, 0)),
            pl.BlockSpec((1, C), lambda n: (0, 0)),
            pl.BlockSpec((1, C), lambda n: (0, 0)),
            pl.BlockSpec((1, C), lambda n: (0, 0)),
            pl.BlockSpec((9 * C, C), lambda n: (0, 0)),
            pl.BlockSpec((1, C), lambda n: (0, 0)),
            pl.BlockSpec((9 * C, C), lambda n: (0, 0)),
            pl.BlockSpec((1, C), lambda n: (0, 0)),
        ],
        out_specs=pl.BlockSpec((1, Ho, Wo, C), lambda n: (n, 0, 0, 0)),
        scratch_shapes=[
            pltpu.VMEM((H, 2, W, 2 * C), bf16),
            pltpu.VMEM((Ho + 2, Wo + 2, Ctot), bf16),
            pltpu.VMEM((Ho + 2, Wo + 2, C), bf16),
        ],
        compiler_params=pltpu.CompilerParams(
            dimension_semantics=("parallel",)),
    )(x, bridge, wup_p, bup_p, wuc_p, buc_p, g_p, bln_p, w1_p, b1_p, w2_p, b2_p)
